# baseline (speedup 1.0000x reference)
import functools

import jax
import jax.numpy as jnp
from jax import lax
from jax.experimental import pallas as pl
from jax.experimental.pallas import tpu as pltpu
from jax.experimental.pallas import tpu_sc as plsc

R, C = 128, 32768
L = 16
NV = C // L
NC, NS = 2, 16
NW = NC * NS
RPW = R // NW
MINI32 = -2147483648


def _lsr(v, s):
    return lax.shift_right_logical(v, jnp.full(v.shape, s, v.dtype))


def _lsl(v, s):
    return lax.shift_left(v, jnp.full(v.shape, s, v.dtype))


def _key(x):
    b = lax.bitcast_convert_type(x, jnp.int32)
    return jnp.where(b < 0, ~b, b | MINI32)


def _unkey(k):
    b = jnp.where(k < 0, k & jnp.int32(0x7FFFFFFF), ~k)
    return lax.bitcast_convert_type(b, jnp.float32)


def _histo_pass(xbuf, hc, hs, shift, width, prefix, lvl):
    nb = 1 << width
    zi = jnp.zeros((L,), jnp.int32)
    zf = jnp.zeros((L,), jnp.float32)

    @plsc.parallel_loop(0, nb // L, unroll=16)
    def _(j):
        hc[pl.ds(j * L, L)] = zi
        hs[pl.ds(j * L, L)] = zf

    ones = jnp.ones((L,), jnp.int32)

    @plsc.parallel_loop(0, NV, unroll=8)
    def _(i):
        x = xbuf[pl.ds(i * L, L)]
        k = _key(x)
        dig = _lsr(k, shift) & (nb - 1)
        if lvl == 0:
            mask = None
        else:
            mask = _lsr(k, shift + width) == prefix
        plsc.addupdate_scatter(hc, [dig], ones, mask=mask)
        plsc.addupdate_scatter(hs, [dig], x, mask=mask)


def _pick16(c_v, s_v, dig_desc, edges_desc, base_k, base_s, k_acc_f, s_acc):
    c_d = lax.rev(c_v, (0,))
    s_d = lax.rev(s_v, (0,))
    sufK = plsc.cumsum(c_d) + base_k
    sufS = plsc.cumsum(s_d) + base_s
    f = (s_acc + sufS) - (k_acc_f + sufK.astype(jnp.float32)) * edges_desc - 1.0
    cond = f >= 0.0
    sel = jnp.max(jnp.where(cond, dig_desc, -1))
    lm = cond & (dig_desc == sel)
    kx = jnp.max(jnp.where(lm, sufK, MINI32))
    sx = jnp.max(jnp.where(lm, sufS, -jnp.inf))
    cb = jnp.max(jnp.where(lm, c_d, MINI32))
    sb = jnp.max(jnp.where(lm, s_d, -jnp.inf))
    return sel, kx, sx, cb, sb


def _edges(iota, prefix, dig_desc, shift):
    return _unkey(_lsl(prefix | dig_desc, shift))


def _scan256(cr, sr, gran, prefix_sh, shift, k_acc_f, s_acc):
    iota = lax.iota(jnp.int32, L)

    @plsc.parallel_loop(0, L, unroll=L, carry=(jnp.zeros((L,), jnp.int32),
                                               jnp.zeros((L,), jnp.float32)))
    def _gtot(jj, carry):
        tc, ts = carry
        idx = iota * L + jj
        tc = tc + plsc.load_gather(cr, [idx])
        ts = ts + plsc.load_gather(sr, [idx])
        return tc, ts

    tot_c, tot_s = _gtot
    blk_desc = (L - 1) - iota
    rc = lax.rev(tot_c, (0,))
    rs = lax.rev(tot_s, (0,))
    sufKb = plsc.cumsum(rc)
    sufSb = plsc.cumsum(rs)
    e_b = _edges(iota, prefix_sh, blk_desc * (L * gran), shift)
    f_b = (s_acc + sufSb) - (k_acc_f + sufKb.astype(jnp.float32)) * e_b - 1.0
    condb = f_b >= 0.0
    bsel = jnp.max(jnp.where(condb, blk_desc, -1))
    lmb = condb & (blk_desc == bsel)
    k_abv = jnp.max(jnp.where(lmb, sufKb - rc, MINI32))
    s_abv = jnp.max(jnp.where(lmb, sufSb - rs, -jnp.inf))

    c_v = cr[pl.ds(bsel * L, L)]
    s_v = sr[pl.ds(bsel * L, L)]
    ent_desc = bsel * L + (L - 1) - iota
    e_f = _edges(iota, prefix_sh, ent_desc * gran, shift)
    return _pick16(c_v, s_v, ent_desc, e_f, k_abv, s_abv, k_acc_f, s_acc)


def _scan_level(hc, hs, tbc, tbs, width, shift, prefix, k_acc, s_acc):
    iota = lax.iota(jnp.int32, L)
    k_acc_f = k_acc.astype(jnp.float32)
    prefix_sh = prefix << width

    if width == 8:
        sel, kx, sx, cb, sb = _scan256(hc, hs, 1, prefix_sh, shift,
                                       k_acc_f, s_acc)
    else:
        @plsc.parallel_loop(0, L, unroll=4)
        def _(v):
            acc_c = jnp.zeros((L,), jnp.int32)
            acc_s = jnp.zeros((L,), jnp.float32)
            for jj in range(L):
                idx = v * 256 + iota * L + jj
                acc_c = acc_c + plsc.load_gather(hc, [idx])
                acc_s = acc_s + plsc.load_gather(hs, [idx])
            tbc[pl.ds(v * L, L)] = acc_c
            tbs[pl.ds(v * L, L)] = acc_s

        bsel, kxB, sxB, cbB, sbB = _scan256(tbc, tbs, L, prefix_sh, shift,
                                            k_acc_f, s_acc)
        base_k = kxB - cbB
        base_s = sxB - sbB
        c_v = hc[pl.ds(bsel * L, L)]
        s_v = hs[pl.ds(bsel * L, L)]
        dig_desc = bsel * L + (L - 1) - iota
        e_f = _edges(iota, prefix_sh, dig_desc, shift)
        sel, kx, sx, cb, sb = _pick16(c_v, s_v, dig_desc, e_f,
                                      base_k, base_s, k_acc_f, s_acc)

    prefix = prefix_sh | sel
    k_acc = k_acc + (kx - cb)
    s_acc = s_acc + (sx - sb)
    return prefix, k_acc, s_acc


@functools.lru_cache(maxsize=1)
def _build():
    mesh = plsc.VectorSubcoreMesh(core_axis_name="c", subcore_axis_name="s",
                                  num_cores=NC, num_subcores=NS)

    @functools.partial(
        pl.kernel,
        out_type=jax.ShapeDtypeStruct((R, C), jnp.float32),
        mesh=mesh,
        compiler_params=pltpu.CompilerParams(needs_layout_passes=False),
        scratch_types=[
            pltpu.VMEM((C,), jnp.float32),
            pltpu.VMEM((C,), jnp.float32),
            pltpu.VMEM((C,), jnp.float32),
            pltpu.VMEM((4096,), jnp.int32),
            pltpu.VMEM((4096,), jnp.float32),
            pltpu.VMEM((256,), jnp.int32),
            pltpu.VMEM((256,), jnp.float32),
            pltpu.SemaphoreType.DMA,
            pltpu.SemaphoreType.DMA,
        ],
    )
    def _sparsemax_sc(in_hbm, out_hbm, xb0, xb1, ob, hc, hs, tbc, tbs,
                      in_sem, out_sem):
        wid = lax.axis_index("s") * NC + lax.axis_index("c")
        base = wid * RPW
        xbufs = (xb0, xb1)

        pltpu.async_copy(in_hbm.at[base], xb0, in_sem)
        for r in range(RPW):
            xb = xbufs[r % 2]
            pltpu.make_async_copy(in_hbm.at[base + r], xb, in_sem).wait()
            if r + 1 < RPW:
                pltpu.async_copy(in_hbm.at[base + r + 1], xbufs[(r + 1) % 2],
                                 in_sem)

            prefix = jnp.int32(0)
            k_acc = jnp.int32(0)
            s_acc = jnp.float32(0.0)
            for lvl, (width, shift) in enumerate(((8, 24), (12, 12),
                                                  (12, 0))):
                _histo_pass(xb, hc, hs, shift, width, prefix, lvl)
                prefix, k_acc, s_acc = _scan_level(hc, hs, tbc, tbs, width,
                                                   shift, prefix, k_acc,
                                                   s_acc)
            tau = (jnp.full((L,), s_acc - 1.0, jnp.float32)
                   / jnp.full((L,), k_acc, jnp.int32).astype(jnp.float32))

            if r >= 1:
                pltpu.make_async_copy(ob, out_hbm.at[base + r - 1],
                                      out_sem).wait()

            @plsc.parallel_loop(0, NV, unroll=8)
            def _(i):
                x = xb[pl.ds(i * L, L)]
                ob[pl.ds(i * L, L)] = jnp.maximum(x - tau, 0.0)

            pltpu.async_copy(ob, out_hbm.at[base + r], out_sem)

        pltpu.make_async_copy(ob, out_hbm.at[base + RPW - 1],
                              out_sem).wait()

    return _sparsemax_sc


def kernel(input):
    return _build()(input)
